# Initial kernel scaffold; baseline (speedup 1.0000x reference)
import jax
import jax.numpy as jnp
from jax.experimental import pallas as pl


def kernel(x, embedding, feedforward_0, feedforward_1, feedforward_2, feedforward_3):
    raise NotImplementedError("write your pallas kernel here")


# speedup vs baseline: 1.5003x; 1.5003x over previous
import jax
import jax.numpy as jnp
from jax import lax
from jax.experimental import pallas as pl
from jax.experimental.pallas import tpu as pltpu
from jax.experimental.pallas import tpu_sc as plsc

EMBED = 512
NC, NS = 2, 16
NW = NC * NS
CH = 64


def _body(table_hbm, idx_hbm, out_hbm, idx_v, rows_v, sem):
    bpw = idx_v.shape[0]
    nchunk = bpw // CH
    wid = lax.axis_index("s") * NC + lax.axis_index("c")
    base = wid * bpw
    pltpu.sync_copy(idx_hbm.at[pl.ds(base, bpw)], idx_v)

    @pl.loop(0, nchunk)
    def _(c):
        off = c * CH
        pltpu.async_copy(
            table_hbm.at[idx_v.at[pl.ds(off, CH)]], rows_v, sem
        ).wait()
        pltpu.sync_copy(rows_v, out_hbm.at[pl.ds(base + off, CH)])


def kernel(x, embedding, feedforward_0, feedforward_1, feedforward_2,
           feedforward_3):
    batch, hist = x.shape
    b = batch * hist
    xf = x.reshape(b).astype(jnp.int32)
    bpw = b // NW

    mesh = plsc.VectorSubcoreMesh(
        core_axis_name="c", subcore_axis_name="s",
        num_cores=NC, num_subcores=NS)
    gather = pl.kernel(
        _body,
        out_type=jax.ShapeDtypeStruct((b, EMBED), jnp.float32),
        mesh=mesh,
        scratch_types=[
            pltpu.VMEM((bpw,), jnp.int32),
            pltpu.VMEM((CH, EMBED), jnp.float32),
            pltpu.SemaphoreType.DMA,
        ],
    )
    out = gather(embedding, xf)
    return out.reshape(batch, hist, EMBED)


# speedup vs baseline: 1.5073x; 1.0046x over previous
import jax
import jax.numpy as jnp
from jax import lax
from jax.experimental import pallas as pl
from jax.experimental.pallas import tpu as pltpu
from jax.experimental.pallas import tpu_sc as plsc

EMBED = 512
NC, NS = 2, 16
NW = NC * NS
CH = 64


def _body(table_hbm, idx_hbm, out_hbm, idx_v, rows_v,
          gsem0, gsem1, ssem0, ssem1):
    bpw = idx_v.shape[0]
    nchunk = bpw // CH
    wid = lax.axis_index("s") * NC + lax.axis_index("c")
    base = wid * bpw
    pltpu.sync_copy(idx_hbm.at[pl.ds(base, bpw)], idx_v)

    gsems = (gsem0, gsem1)
    ssems = (ssem0, ssem1)

    def gstart(c, b):
        pltpu.async_copy(
            table_hbm.at[idx_v.at[pl.ds(c * CH, CH)]], rows_v.at[b],
            gsems[b])

    def gwait(c, b):
        pltpu.make_async_copy(
            table_hbm.at[idx_v.at[pl.ds(c * CH, CH)]], rows_v.at[b],
            gsems[b]).wait()

    def sstart(c, b):
        pltpu.async_copy(
            rows_v.at[b], out_hbm.at[pl.ds(base + c * CH, CH)], ssems[b])

    def swait(c, b):
        pltpu.make_async_copy(
            rows_v.at[b], out_hbm.at[pl.ds(base + c * CH, CH)],
            ssems[b]).wait()

    gstart(0, 0)
    gstart(1, 1)
    gwait(0, 0)
    sstart(0, 0)

    @pl.loop(2, nchunk, step=2)
    def _(g):
        for b in (0, 1):
            c = g + b
            swait(c - 2, b)
            gstart(c, b)
            gwait(c - 1, 1 - b)
            sstart(c - 1, 1 - b)

    gwait(nchunk - 1, 1)
    sstart(nchunk - 1, 1)
    swait(nchunk - 2, 0)
    swait(nchunk - 1, 1)


def kernel(x, embedding, feedforward_0, feedforward_1, feedforward_2,
           feedforward_3):
    batch, hist = x.shape
    b = batch * hist
    xf = x.reshape(b).astype(jnp.int32)
    bpw = b // NW

    mesh = plsc.VectorSubcoreMesh(
        core_axis_name="c", subcore_axis_name="s",
        num_cores=NC, num_subcores=NS)
    gather = pl.kernel(
        _body,
        out_type=jax.ShapeDtypeStruct((b, EMBED), jnp.float32),
        mesh=mesh,
        scratch_types=[
            pltpu.VMEM((bpw,), jnp.int32),
            pltpu.VMEM((2, CH, EMBED), jnp.float32),
            pltpu.SemaphoreType.DMA,
            pltpu.SemaphoreType.DMA,
            pltpu.SemaphoreType.DMA,
            pltpu.SemaphoreType.DMA,
        ],
    )
    out = gather(embedding, xf)
    return out.reshape(batch, hist, EMBED)


# speedup vs baseline: 3.4860x; 2.3128x over previous
import jax
import jax.numpy as jnp
from jax import lax
from jax.experimental import pallas as pl
from jax.experimental.pallas import tpu as pltpu
from jax.experimental.pallas import tpu_sc as plsc

EMBED = 512
NC, NS = 2, 16
NW = NC * NS
CB = 64


def _body(table_hbm, idx_hbm, out_hbm, idx_v, rows_v,
          gsem0, gsem1, ssem0, ssem1):
    hist, bpw = idx_v.shape
    nchunk = (hist * bpw) // CB
    wid = lax.axis_index("s") * NC + lax.axis_index("c")
    b0 = wid * bpw
    pltpu.sync_copy(idx_hbm.at[:, pl.ds(b0, bpw)], idx_v)

    gsems = (gsem0, gsem1)
    ssems = (ssem0, ssem1)

    def gstart(h, half, b):
        pltpu.async_copy(
            table_hbm.at[idx_v.at[h].at[pl.ds(half * CB, CB)]],
            rows_v.at[b], gsems[b])

    def gwait(h, half, b):
        pltpu.make_async_copy(
            table_hbm.at[idx_v.at[h].at[pl.ds(half * CB, CB)]],
            rows_v.at[b], gsems[b]).wait()

    def sstart(h, half, b):
        pltpu.async_copy(
            rows_v.at[b], out_hbm.at[h].at[pl.ds(b0 + half * CB, CB)],
            ssems[b])

    def swait(h, half, b):
        pltpu.make_async_copy(
            rows_v.at[b], out_hbm.at[h].at[pl.ds(b0 + half * CB, CB)],
            ssems[b]).wait()

    nhalf = bpw // CB

    gstart(0, 0, 0)
    gstart(0, 1, 1)
    gwait(0, 0, 0)
    sstart(0, 0, 0)

    @pl.loop(1, hist)
    def _(h):
        for half in range(nhalf):
            b = half
            swait(h - 1, half, b)
            gstart(h, half, b)
            ph, phalf = (h, 0) if half == 1 else (h - 1, 1)
            gwait(ph, phalf, 1 - b)
            sstart(ph, phalf, 1 - b)

    gwait(hist - 1, 1, 1)
    sstart(hist - 1, 1, 1)
    swait(hist - 1, 0, 0)
    swait(hist - 1, 1, 1)


def kernel(x, embedding, feedforward_0, feedforward_1, feedforward_2,
           feedforward_3):
    batch, hist = x.shape
    xt = jnp.swapaxes(x.astype(jnp.int32), 0, 1)
    bpw = batch // NW

    mesh = plsc.VectorSubcoreMesh(
        core_axis_name="c", subcore_axis_name="s",
        num_cores=NC, num_subcores=NS)
    gather = pl.kernel(
        _body,
        out_type=jax.ShapeDtypeStruct((hist, batch, EMBED), jnp.float32),
        mesh=mesh,
        scratch_types=[
            pltpu.VMEM((hist, bpw), jnp.int32),
            pltpu.VMEM((2, CB, EMBED), jnp.float32),
            pltpu.SemaphoreType.DMA,
            pltpu.SemaphoreType.DMA,
            pltpu.SemaphoreType.DMA,
            pltpu.SemaphoreType.DMA,
        ],
    )
    out = gather(embedding, xt)
    return jnp.transpose(out, (1, 0, 2))
